# baseline (speedup 1.0000x reference)
import functools

import jax
import jax.numpy as jnp
from jax.experimental import pallas as pl
from jax.experimental.pallas import tpu as pltpu

_EPS = 1e-07


def _inv3(m):
    m00 = m[..., 0, 0]; m01 = m[..., 0, 1]; m02 = m[..., 0, 2]
    m10 = m[..., 1, 0]; m11 = m[..., 1, 1]; m12 = m[..., 1, 2]
    m20 = m[..., 2, 0]; m21 = m[..., 2, 1]; m22 = m[..., 2, 2]
    det = (m00 * (m11 * m22 - m12 * m21)
           - m01 * (m10 * m22 - m12 * m20)
           + m02 * (m10 * m21 - m11 * m20))
    cof = jnp.stack([
        jnp.stack([m11 * m22 - m12 * m21, m02 * m21 - m01 * m22, m01 * m12 - m02 * m11], -1),
        jnp.stack([m12 * m20 - m10 * m22, m00 * m22 - m02 * m20, m02 * m10 - m00 * m12], -1),
        jnp.stack([m10 * m21 - m11 * m20, m01 * m20 - m00 * m21, m00 * m11 - m01 * m10], -1),
    ], -2)
    return cof / (det + _EPS)[..., None, None]


def _cayley(g):
    eye = jnp.eye(3, dtype=g.dtype)
    idx = jnp.array([[0, 1, 2], [3, 4, 5], [6, 7, 4]])
    gk = g[:, :, idx, :]
    a_, b_, c_ = gk[..., 0], gk[..., 1], gk[..., 2]
    z = jnp.zeros_like(a_)
    skew = jnp.stack([
        jnp.stack([z, a_, b_], -1),
        jnp.stack([-a_, z, c_], -1),
        jnp.stack([-b_, -c_, z], -1),
    ], -2)
    num = skew.at[:, :, 1, 1].set(eye)
    den = (eye - skew).at[:, :, 1, 1].set(eye)
    return _inv3(den) @ (eye + num)


def _conv_body(u_ref, n_ref, e_ref, wi_ref, wr_ref, o_ref, s_ref, *, rb, w, nr, co):
    wp = w + 2
    j = pl.program_id(1)

    ub = u_ref[0]
    v = jnp.concatenate([ub[1:], n_ref[0, :1]], axis=0)
    f32 = jnp.float32
    zc = jnp.zeros((rb, 1, 9), f32)

    def shl(y):
        return jnp.concatenate([y[:, 1:, :], zc], axis=1)

    def shr(y):
        return jnp.concatenate([zc, y[:, :w - 1, :]], axis=1)

    xc = jnp.concatenate([ub, shl(ub), shr(v), v, shl(v)], axis=2)
    acc = jnp.dot(xc.reshape(rb * w, 45), wi_ref[...],
                  preferred_element_type=f32).reshape(rb, w, co)
    o_ref[0, :, 1:w + 1, :] = acc

    def ring(side, w0):
        xe = e_ref[0, side]
        terms = []
        for t in range(3):
            y = jax.lax.dot_general(
                xe, wr_ref[w0 + t], (((2,), (1,)), ((0,), (0,))),
                preferred_element_type=f32)
            y = jnp.sum(y, axis=0)
            parts = [jnp.zeros((t, co), f32), y, jnp.zeros((2 - t, co), f32)]
            terms.append(jnp.concatenate([p for p in parts if p.shape[0]], axis=0))
        return terms[0] + terms[1] + terms[2]

    @pl.when(j == 0)
    def _():
        s_ref[0] = ring(2, 6)
        s_ref[1] = ring(3, 9)

    for jj in range(nr):
        @pl.when(j == jj)
        def _(jj=jj):
            o_ref[0, :, 0, :] = s_ref[0, jj * rb:(jj + 1) * rb, :]
            o_ref[0, :, wp - 1, :] = s_ref[1, jj * rb:(jj + 1) * rb, :]

    @pl.when(j == 0)
    def _():
        o_ref[0, 0, :, :] = ring(0, 0)

    @pl.when(j == nr - 1)
    def _():
        o_ref[0, rb - 1, :, :] = ring(1, 3)


def kernel(x, g):
    b, ci_n, h, w = x.shape[:4]
    o_n = g.shape[0]
    i_n = g.shape[1]
    hp, wp = h + 2, w + 2
    ci = i_n * 9
    co = o_n * 9

    gmat = _cayley(g)

    def m_full(a, t):
        gab = gmat[:, :, a, t]
        return jnp.einsum('oipq,oits->iqsopt', gab, gab).reshape(ci, co)

    glast = gmat[:, -1]

    def m_last(a, t):
        gab = glast[:, a, t]
        return jnp.einsum('opq,ots->qsopt', gab, gab).reshape(9, co)

    eye9 = jnp.concatenate([jnp.eye(9, dtype=x.dtype)] * o_n, axis=1)
    w_int = jnp.concatenate(
        [eye9, m_last(1, 0), m_last(0, 2), m_last(0, 1), m_last(0, 0)], axis=0)
    w_ring = jnp.stack(
        [m_full(0, t) for t in range(3)] + [m_full(2, t) for t in range(3)]
        + [m_full(a, 0) for a in range(3)] + [m_full(a, 2) for a in range(3)]
    ).reshape(12, i_n, 9, co)

    xl = x[:, -1].reshape(b, h, w, 9)
    u = jnp.pad(xl, ((0, 0), (1, 1), (0, 0), (0, 0)))

    edges = jnp.stack([
        x[:, :, 0].reshape(b, i_n, w, 9),
        x[:, :, h - 1].reshape(b, i_n, w, 9),
        x[:, :, :, 0].reshape(b, i_n, w, 9),
        x[:, :, :, w - 1].reshape(b, i_n, w, 9),
    ], axis=1)

    nr = 5 if hp % 5 == 0 else 1
    rb = hp // nr

    out_ch = pl.pallas_call(
        functools.partial(_conv_body, rb=rb, w=w, nr=nr, co=co),
        grid=(b, nr),
        in_specs=[
            pl.BlockSpec((1, rb, w, 9), lambda bi, j: (bi, j, 0, 0)),
            pl.BlockSpec((1, rb, w, 9),
                         lambda bi, j: (bi, jnp.minimum(j + 1, nr - 1), 0, 0)),
            pl.BlockSpec((1, 4, i_n, w, 9), lambda bi, j: (bi, 0, 0, 0, 0)),
            pl.BlockSpec((45, co), lambda bi, j: (0, 0)),
            pl.BlockSpec((12, i_n, 9, co), lambda bi, j: (0, 0, 0, 0)),
        ],
        out_specs=pl.BlockSpec((1, rb, wp, co), lambda bi, j: (bi, j, 0, 0)),
        out_shape=jax.ShapeDtypeStruct((b, hp, wp, co), x.dtype),
        scratch_shapes=[pltpu.VMEM((2, hp, co), jnp.float32)],
        compiler_params=pltpu.CompilerParams(
            dimension_semantics=("parallel", "arbitrary"),
        ),
    )(u, u, edges, w_int, w_ring)

    return (out_ch.reshape(b, hp, wp, o_n, 9)
            .transpose(0, 3, 1, 2, 4)
            .reshape(b, o_n, hp, wp, 3, 3))


# speedup vs baseline: 1.0378x; 1.0378x over previous
import functools

import jax
import jax.numpy as jnp
from jax.experimental import pallas as pl
from jax.experimental.pallas import tpu as pltpu

_EPS = 1e-07


def _inv3(m):
    m00 = m[..., 0, 0]; m01 = m[..., 0, 1]; m02 = m[..., 0, 2]
    m10 = m[..., 1, 0]; m11 = m[..., 1, 1]; m12 = m[..., 1, 2]
    m20 = m[..., 2, 0]; m21 = m[..., 2, 1]; m22 = m[..., 2, 2]
    det = (m00 * (m11 * m22 - m12 * m21)
           - m01 * (m10 * m22 - m12 * m20)
           + m02 * (m10 * m21 - m11 * m20))
    cof = jnp.stack([
        jnp.stack([m11 * m22 - m12 * m21, m02 * m21 - m01 * m22, m01 * m12 - m02 * m11], -1),
        jnp.stack([m12 * m20 - m10 * m22, m00 * m22 - m02 * m20, m02 * m10 - m00 * m12], -1),
        jnp.stack([m10 * m21 - m11 * m20, m01 * m20 - m00 * m21, m00 * m11 - m01 * m10], -1),
    ], -2)
    return cof / (det + _EPS)[..., None, None]


def _cayley(g):
    eye = jnp.eye(3, dtype=g.dtype)
    idx = jnp.array([[0, 1, 2], [3, 4, 5], [6, 7, 4]])
    gk = g[:, :, idx, :]
    a_, b_, c_ = gk[..., 0], gk[..., 1], gk[..., 2]
    z = jnp.zeros_like(a_)
    skew = jnp.stack([
        jnp.stack([z, a_, b_], -1),
        jnp.stack([-a_, z, c_], -1),
        jnp.stack([-b_, -c_, z], -1),
    ], -2)
    num = skew.at[:, :, 1, 1].set(eye)
    den = (eye - skew).at[:, :, 1, 1].set(eye)
    return _inv3(den) @ (eye + num)


def _conv_body(u_ref, n_ref, e_ref, wi_ref, wr_ref, o_ref, s_ref, *, rb, w, nr, co):
    wp = w + 2
    j = pl.program_id(1)

    ub = u_ref[0]
    v = jnp.concatenate([ub[1:], n_ref[0, :1]], axis=0)
    f32 = jnp.float32
    zc = jnp.zeros((rb, 1, 9), f32)

    def shl(y):
        return jnp.concatenate([y[:, 1:, :], zc], axis=1)

    def shr(y):
        return jnp.concatenate([zc, y[:, :w - 1, :]], axis=1)

    xc = jnp.concatenate([ub, shl(ub), shr(v), v, shl(v)], axis=2)
    acc = jnp.dot(xc.reshape(rb * w, 45), wi_ref[...],
                  preferred_element_type=f32).reshape(rb, w, co)
    o_ref[0, :, 1:w + 1, :] = acc

    def ring(side, w0):
        xe = e_ref[0, side]
        terms = []
        for t in range(3):
            y = jnp.dot(xe, wr_ref[w0 + t], preferred_element_type=f32)
            parts = [jnp.zeros((t, co), f32), y, jnp.zeros((2 - t, co), f32)]
            terms.append(jnp.concatenate([p for p in parts if p.shape[0]], axis=0))
        return terms[0] + terms[1] + terms[2]

    @pl.when(j == 0)
    def _():
        s_ref[0] = ring(2, 6)
        s_ref[1] = ring(3, 9)

    for jj in range(nr):
        @pl.when(j == jj)
        def _(jj=jj):
            o_ref[0, :, 0, :] = s_ref[0, jj * rb:(jj + 1) * rb, :]
            o_ref[0, :, wp - 1, :] = s_ref[1, jj * rb:(jj + 1) * rb, :]

    @pl.when(j == 0)
    def _():
        o_ref[0, 0, :, :] = ring(0, 0)

    @pl.when(j == nr - 1)
    def _():
        o_ref[0, rb - 1, :, :] = ring(1, 3)


def kernel(x, g):
    b, ci_n, h, w = x.shape[:4]
    o_n = g.shape[0]
    i_n = g.shape[1]
    hp, wp = h + 2, w + 2
    ci = i_n * 9
    co = o_n * 9

    gmat = _cayley(g)

    def m_full(a, t):
        gab = gmat[:, :, a, t]
        return jnp.einsum('oipq,oits->iqsopt', gab, gab).reshape(ci, co)

    glast = gmat[:, -1]

    def m_last(a, t):
        gab = glast[:, a, t]
        return jnp.einsum('opq,ots->qsopt', gab, gab).reshape(9, co)

    eye9 = jnp.concatenate([jnp.eye(9, dtype=x.dtype)] * o_n, axis=1)
    w_int = jnp.concatenate(
        [eye9, m_last(1, 0), m_last(0, 2), m_last(0, 1), m_last(0, 0)], axis=0)
    w_ring = jnp.stack(
        [m_full(0, t) for t in range(3)] + [m_full(2, t) for t in range(3)]
        + [m_full(a, 0) for a in range(3)] + [m_full(a, 2) for a in range(3)]
    )

    xl = x[:, -1].reshape(b, h, w, 9)
    u = jnp.pad(xl, ((0, 0), (1, 1), (0, 0), (0, 0)))

    def edge_rows(sl):
        return sl.transpose(0, 2, 1, 3, 4).reshape(b, w, ci)

    edges = jnp.stack([
        edge_rows(x[:, :, 0]), edge_rows(x[:, :, h - 1]),
        edge_rows(x[:, :, :, 0]), edge_rows(x[:, :, :, w - 1]),
    ], axis=1)

    nr = 2 if hp % 2 == 0 else 1
    rb = hp // nr

    out_ch = pl.pallas_call(
        functools.partial(_conv_body, rb=rb, w=w, nr=nr, co=co),
        grid=(b, nr),
        in_specs=[
            pl.BlockSpec((1, rb, w, 9), lambda bi, j: (bi, j, 0, 0)),
            pl.BlockSpec((1, rb, w, 9),
                         lambda bi, j: (bi, jnp.minimum(j + 1, nr - 1), 0, 0)),
            pl.BlockSpec((1, 4, w, ci), lambda bi, j: (bi, 0, 0, 0)),
            pl.BlockSpec((45, co), lambda bi, j: (0, 0)),
            pl.BlockSpec((12, ci, co), lambda bi, j: (0, 0, 0)),
        ],
        out_specs=pl.BlockSpec((1, rb, wp, co), lambda bi, j: (bi, j, 0, 0)),
        out_shape=jax.ShapeDtypeStruct((b, hp, wp, co), x.dtype),
        scratch_shapes=[pltpu.VMEM((2, hp, co), jnp.float32)],
        compiler_params=pltpu.CompilerParams(
            dimension_semantics=("parallel", "arbitrary"),
            vmem_limit_bytes=52 * 1024 * 1024,
        ),
    )(u, u, edges, w_int, w_ring)

    return (out_ch.reshape(b, hp, wp, o_n, 9)
            .transpose(0, 3, 1, 2, 4)
            .reshape(b, o_n, hp, wp, 3, 3))
